# baseline (speedup 1.0000x reference)
import jax
import jax.numpy as jnp
from jax.experimental import pallas as pl

_BATCH_BLOCK = 8


def _add_table_kernel(x_ref, t_ref, o_ref):
    o_ref[...] = x_ref[...] + t_ref[...][None, :, :]


def kernel(encoded_patches, position_table):
    batch, num_patches, dim = encoded_patches.shape
    grid = (batch // _BATCH_BLOCK,)
    return pl.pallas_call(
        _add_table_kernel,
        grid=grid,
        in_specs=[
            pl.BlockSpec((_BATCH_BLOCK, num_patches, dim), lambda i: (i, 0, 0)),
            pl.BlockSpec((num_patches, dim), lambda i: (0, 0)),
        ],
        out_specs=pl.BlockSpec((_BATCH_BLOCK, num_patches, dim), lambda i: (i, 0, 0)),
        out_shape=jax.ShapeDtypeStruct(encoded_patches.shape, encoded_patches.dtype),
    )(encoded_patches, position_table)


# speedup vs baseline: 1.0073x; 1.0073x over previous
import jax
import jax.numpy as jnp
from jax.experimental import pallas as pl

_BATCH_BLOCK = 16


def _add_table_kernel(x_ref, t_ref, o_ref):
    o_ref[...] = x_ref[...] + t_ref[...][None, :, :]


def kernel(encoded_patches, position_table):
    batch, num_patches, dim = encoded_patches.shape
    grid = (batch // _BATCH_BLOCK,)
    return pl.pallas_call(
        _add_table_kernel,
        grid=grid,
        in_specs=[
            pl.BlockSpec((_BATCH_BLOCK, num_patches, dim), lambda i: (i, 0, 0)),
            pl.BlockSpec((num_patches, dim), lambda i: (0, 0)),
        ],
        out_specs=pl.BlockSpec((_BATCH_BLOCK, num_patches, dim), lambda i: (i, 0, 0)),
        out_shape=jax.ShapeDtypeStruct(encoded_patches.shape, encoded_patches.dtype),
    )(encoded_patches, position_table)
